# baseline (speedup 1.0000x reference)
import functools
import numpy as np
import jax
import jax.numpy as jnp
from jax import lax
from jax.experimental import pallas as pl
from jax.experimental.pallas import tpu as pltpu
from jax.experimental.pallas import tpu_sc as plsc

MAXF = 1024
D_EMB = 256
N_FEAT = 10
D_OUT = D_EMB + 2 * N_FEAT
D_PAD = 288
D_TAIL = 48


def _fourier_table_np():
    powers = (2.0 ** np.arange(N_FEAT)).astype(np.float32)
    coefs = (powers * (np.pi / MAXF)).astype(np.float32)
    raw = np.arange(MAXF, dtype=np.float32)[:, None] * coefs[None, :]
    return np.concatenate([np.sin(raw), np.cos(raw)], axis=1).astype(np.float32)


_FTAB = _fourier_table_np()


@functools.lru_cache(maxsize=None)
def _build_sc_call(batch, nframes):
    NC, NS, L = 2, 16, 16
    NW = NC * NS
    BW = batch // NW
    CB = 64
    NCH = BW // CB

    mesh = plsc.VectorSubcoreMesh(
        core_axis_name="c", subcore_axis_name="s",
        num_cores=NC, num_subcores=NS)

    big_ty = pltpu.VMEM((CB, D_PAD), jnp.float32)
    tail_ty = pltpu.VMEM((CB, D_TAIL), jnp.float32)
    panel = jax.ShapeDtypeStruct((batch, 128), jnp.float32)

    @functools.partial(
        pl.kernel,
        out_type=(panel, panel, panel, panel, panel),
        mesh=mesh,
        compiler_params=pltpu.CompilerParams(use_tc_tiling_on_sc=False),
        scratch_types=[
            pltpu.VMEM((BW,), jnp.int32),
            pltpu.VMEM((BW,), jnp.int32),
            pltpu.VMEM((BW,), jnp.int32),
            pltpu.VMEM((BW,), jnp.int32),
            pltpu.VMEM((BW,), jnp.int32),
            big_ty, big_ty,
            big_ty, big_ty,
            tail_ty, tail_ty,
            pltpu.SemaphoreType.DMA, pltpu.SemaphoreType.DMA,
            pltpu.SemaphoreType.DMA, pltpu.SemaphoreType.DMA,
            pltpu.SemaphoreType.DMA, pltpu.SemaphoreType.DMA,
            pltpu.SemaphoreType.DMA, pltpu.SemaphoreType.DMA,
        ],
    )
    def sc_call(t_hbm, tabe_hbm, taboa_hbm, tabob_hbm,
                p0_hbm, p1_hbm, p2_hbm, p3_hbm, p4_hbm,
                t0_v, t1_v, t2_v, de_v, do_v,
                be0, be1, ba0, ba1, bb0, bb1,
                sge0, sge1, sga0, sga1, sgb0, sgb1, sw0, sw1):
        wid = lax.axis_index("s") * NC + lax.axis_index("c")
        b0 = wid * BW
        pltpu.sync_copy(t_hbm.at[pl.ds(b0, BW)], t0_v)
        pltpu.sync_copy(t_hbm.at[pl.ds(batch + b0, BW)], t1_v)
        pltpu.sync_copy(t_hbm.at[pl.ds(2 * batch + b0, BW)], t2_v)

        def diff_body(g, carry):
            s = pl.ds(g * L, L)
            de_v[s] = t1_v[s] - t0_v[s]
            do_v[s] = t2_v[s] - t1_v[s]
            return carry

        lax.fori_loop(0, BW // L, diff_body, 0)

        lane = lax.iota(jnp.int32, L)
        in_lo = lane < 4

        bufe = (be0, be1)
        bufa = (ba0, ba1)
        bufb = (bb0, bb1)
        sge = (sge0, sge1)
        sga = (sga0, sga1)
        sgb = (sgb0, sgb1)
        sw = (sw0, sw1)

        gh = [None] * NCH
        wh = [None] * NCH

        def start_gathers(c):
            i = c % 2
            idx_e = de_v.at[pl.ds(c * CB, CB)]
            idx_o = do_v.at[pl.ds(c * CB, CB)]
            gh[c] = (
                pltpu.async_copy(tabe_hbm.at[idx_e], bufe[i], sge[i]),
                pltpu.async_copy(taboa_hbm.at[idx_o], bufa[i], sga[i]),
                pltpu.async_copy(tabob_hbm.at[idx_o], bufb[i], sgb[i]),
            )

        start_gathers(0)
        for c in range(NCH):
            i = c % 2
            if c + 1 < NCH:
                if c >= 1:
                    for h in wh[c - 1]:
                        h.wait()
                start_gathers(c + 1)
            for g in gh[c]:
                g.wait()

            def fix_body(j, carry):
                bufa[i][j, pl.ds(0, L)] = bufe[i][j, pl.ds(256, L)]
                ve2 = bufe[i][j, pl.ds(272, L)]
                cur = bufa[i][j, pl.ds(L, L)]
                bufa[i][j, pl.ds(L, L)] = jnp.where(in_lo, ve2, cur)
                return carry

            lax.fori_loop(0, CB, fix_body, 0)

            rows0 = pl.ds(b0 + c * CB, CB)
            wh[c] = (
                pltpu.async_copy(bufe[i].at[:, pl.ds(0, 128)],
                                 p0_hbm.at[rows0], sw[i]),
                pltpu.async_copy(bufe[i].at[:, pl.ds(128, 128)],
                                 p1_hbm.at[rows0], sw[i]),
                pltpu.async_copy(bufa[i].at[:, pl.ds(0, 128)],
                                 p2_hbm.at[rows0], sw[i]),
                pltpu.async_copy(bufa[i].at[:, pl.ds(128, 128)],
                                 p3_hbm.at[rows0], sw[i]),
                pltpu.async_copy(bufb[i].at[:, pl.ds(0, 40)],
                                 p4_hbm.at[rows0, pl.ds(0, 40)], sw[i]),
            )

        for c in (NCH - 2, NCH - 1):
            for h in wh[c]:
                h.wait()

    return sc_call


def kernel(t, embed_table):
    batch, nframes = t.shape
    ftab = jnp.asarray(_FTAB)
    aug = jnp.concatenate([embed_table, ftab], axis=1)
    z20 = jnp.zeros((MAXF, 20), jnp.float32)
    tab_e = jnp.concatenate([aug, z20[:, :12]], axis=1)
    tab_oa = jnp.concatenate([z20, aug[:, :236], z20, z20[:, :12]], axis=1)
    tab_ob = jnp.concatenate([aug[:, 236:], z20[:, :8]], axis=1)
    half = batch // 2
    call = _build_sc_call(half, nframes)
    out = jnp.zeros((batch, 2 * D_OUT), jnp.float32)
    for h in range(2):
        th = t[h * half:(h + 1) * half]
        panels = call(th.T.reshape(-1), tab_e, tab_oa, tab_ob)
        for p, arr in enumerate(panels):
            w = 40 if p == 4 else 128
            out = lax.dynamic_update_slice(out, arr[:, :w],
                                           (h * half, 128 * p))
    return out


# speedup vs baseline: 1.1521x; 1.1521x over previous
import functools
import numpy as np
import jax
import jax.numpy as jnp
from jax import lax
from jax.experimental import pallas as pl
from jax.experimental.pallas import tpu as pltpu
from jax.experimental.pallas import tpu_sc as plsc

MAXF = 1024
D_EMB = 256
N_FEAT = 10
D_OUT = D_EMB + 2 * N_FEAT


def _fourier_table_np():
    powers = (2.0 ** np.arange(N_FEAT)).astype(np.float32)
    coefs = (powers * (np.pi / MAXF)).astype(np.float32)
    raw = np.arange(MAXF, dtype=np.float32)[:, None] * coefs[None, :]
    return np.concatenate([np.sin(raw), np.cos(raw)], axis=1).astype(np.float32)


_FTAB = _fourier_table_np()

_WIDTHS = (128, 128, 32, 128, 128, 48)


@functools.lru_cache(maxsize=None)
def _build_sc_call(batch, nframes):
    NC, NS, L = 2, 16, 16
    NW = NC * NS
    BW = batch // NW
    CB = 64
    NCH = BW // CB

    mesh = plsc.VectorSubcoreMesh(
        core_axis_name="c", subcore_axis_name="s",
        num_cores=NC, num_subcores=NS)

    panel = jax.ShapeDtypeStruct((batch, 128), jnp.float32)
    bufs = [pltpu.VMEM((CB, w), jnp.float32)
            for w in _WIDTHS for _ in (0, 1)]
    sems = [pltpu.SemaphoreType.DMA] * 14

    @functools.partial(
        pl.kernel,
        out_type=(panel, panel, panel, panel, panel),
        mesh=mesh,
        compiler_params=pltpu.CompilerParams(use_tc_tiling_on_sc=False),
        scratch_types=[
            pltpu.VMEM((BW,), jnp.int32),
            pltpu.VMEM((BW,), jnp.int32),
            pltpu.VMEM((BW,), jnp.int32),
            pltpu.VMEM((BW,), jnp.int32),
            pltpu.VMEM((BW,), jnp.int32),
        ] + bufs + sems,
    )
    def sc_call(t_hbm, te0_h, te1_h, te2_h, to2_h, to3_h, to4_h,
                p0_h, p1_h, p2_h, p3_h, p4_h,
                t0_v, t1_v, t2_v, de_v, do_v, *bufsem):
        buf = [(bufsem[2 * k], bufsem[2 * k + 1]) for k in range(6)]
        gsem = [(bufsem[12 + 2 * k], bufsem[12 + 2 * k + 1])
                for k in range(6)]
        wsem = (bufsem[24], bufsem[25])
        tabs = (te0_h, te1_h, te2_h, to2_h, to3_h, to4_h)
        outs = (p0_h, p1_h, p2_h, p3_h, p4_h)

        wid = lax.axis_index("s") * NC + lax.axis_index("c")
        b0 = wid * BW
        pltpu.sync_copy(t_hbm.at[pl.ds(b0, BW)], t0_v)
        pltpu.sync_copy(t_hbm.at[pl.ds(batch + b0, BW)], t1_v)
        pltpu.sync_copy(t_hbm.at[pl.ds(2 * batch + b0, BW)], t2_v)

        def diff_body(g, carry):
            s = pl.ds(g * L, L)
            de_v[s] = t1_v[s] - t0_v[s]
            do_v[s] = t2_v[s] - t1_v[s]
            return carry

        lax.fori_loop(0, BW // L, diff_body, 0)

        lane = lax.iota(jnp.int32, L)
        in_lo = lane < 4

        gh = [None] * NCH
        wh = [None] * NCH

        def start_gathers(c):
            i = c % 2
            idx_e = de_v.at[pl.ds(c * CB, CB)]
            idx_o = do_v.at[pl.ds(c * CB, CB)]
            gh[c] = tuple(
                pltpu.async_copy(
                    tabs[k].at[idx_e if k < 3 else idx_o],
                    buf[k][i], gsem[k][i])
                for k in range(6))

        start_gathers(0)
        for c in range(NCH):
            i = c % 2
            if c + 1 < NCH:
                if c >= 1:
                    for h in wh[c - 1]:
                        h.wait()
                start_gathers(c + 1)
            for g in gh[c]:
                g.wait()

            be2 = buf[2][i]
            bo2 = buf[3][i]

            def fix_body(j, carry):
                bo2[j, pl.ds(0, L)] = be2[j, pl.ds(0, L)]
                ve2 = be2[j, pl.ds(L, L)]
                cur = bo2[j, pl.ds(L, L)]
                bo2[j, pl.ds(L, L)] = jnp.where(in_lo, ve2, cur)
                return carry

            lax.fori_loop(0, CB, fix_body, 0)

            rows0 = pl.ds(b0 + c * CB, CB)
            srcs = (buf[0][i], buf[1][i], bo2, buf[4][i], buf[5][i])
            wh[c] = tuple(
                pltpu.async_copy(
                    srcs[p] if p < 4 else srcs[p].at[:, pl.ds(0, 40)],
                    outs[p].at[rows0] if p < 4
                    else outs[p].at[rows0, pl.ds(0, 40)],
                    wsem[i])
                for p in range(5))

        for c in (NCH - 2, NCH - 1):
            for h in wh[c]:
                h.wait()

    return sc_call


def kernel(t, embed_table):
    batch, nframes = t.shape
    ftab = jnp.asarray(_FTAB)
    aug = jnp.concatenate([embed_table, ftab], axis=1)
    z20 = jnp.zeros((MAXF, 20), jnp.float32)
    t_e0 = aug[:, 0:128]
    t_e1 = aug[:, 128:256]
    t_e2 = jnp.concatenate([aug[:, 256:276], z20[:, :12]], axis=1)
    t_o2 = jnp.concatenate([z20, aug[:, 0:108]], axis=1)
    t_o3 = aug[:, 108:236]
    t_o4 = jnp.concatenate([aug[:, 236:276], z20[:, :8]], axis=1)
    p0, p1, p2, p3, p4 = _build_sc_call(batch, nframes)(
        t.T.reshape(-1), t_e0, t_e1, t_e2, t_o2, t_o3, t_o4)
    return jnp.concatenate([p0, p1, p2, p3, p4[:, :40]], axis=1)
